# Initial kernel scaffold; baseline (speedup 1.0000x reference)
import jax
import jax.numpy as jnp
from jax.experimental import pallas as pl


def kernel(embedding_features_per_residue, embedding_features_per_sequence, sequence_features, edge_index, edge_attr, batch, W1, bc1, g1, be1, W2, bc2, g2, be2, fc1W, fc1b, g4, be4, fc2W, fc2b, g5, be5, fc3W, fc3b, g6, be6):
    raise NotImplementedError("write your pallas kernel here")


# speedup vs baseline: 5.5434x; 5.5434x over previous
import functools

import jax
import jax.numpy as jnp
from jax import lax
from jax.experimental import pallas as pl
from jax.experimental.pallas import tpu as pltpu
from jax.experimental.pallas import tpu_sc as plsc

N = 10000
E = 80000
G = 64
D_IN = 1280
H = 512
N_PAD = 10112
E_PAD = 81920
NSTEP = 40
EPW_D = E_PAD // 32
ROWS_PER_SUB = N_PAD // 16
RB = 512
NRB = 20
BN_EPS = 1e-5


def _sc_mesh():
    return plsc.VectorSubcoreMesh(core_axis_name="c", subcore_axis_name="s")


def _sc_deg_body(dst_hbm, out_hbm, dst_v, hist_v):
    c = lax.axis_index("c")
    s = lax.axis_index("s")
    w = s * 2 + c
    pltpu.sync_copy(dst_hbm.at[w], dst_v)

    def zero(i, _):
        hist_v[pl.ds(i * 16, 16)] = jnp.zeros((16,), jnp.float32)
        return 0

    lax.fori_loop(0, N_PAD // 16, zero, 0)
    ones = jnp.ones((16,), jnp.float32)

    def add(i, _):
        r = i // 8
        k = i % 8
        idx = dst_v[r, pl.ds(k * 16, 16)]
        plsc.addupdate_scatter(hist_v, [idx], ones)
        return 0

    lax.fori_loop(0, EPW_D // 16, add, 0)
    pltpu.sync_copy(hist_v, out_hbm.at[w, 0])


def _sc_deg(dst32):
    f = pl.kernel(
        _sc_deg_body,
        out_type=jax.ShapeDtypeStruct((32, 1, N_PAD), jnp.float32),
        mesh=_sc_mesh(),
        compiler_params=pltpu.CompilerParams(needs_layout_passes=False),
        scratch_types=[
            pltpu.VMEM((EPW_D // 128, 128), jnp.int32),
            pltpu.VMEM((N_PAD,), jnp.float32),
        ],
    )
    return f(dst32)


def _sc_agg_body(h2d_hbm, src_hbm, dst_hbm, out_hbm,
                 src_v, dst_v, gix_v, rows_v, zero_v, acc, sem):
    c = lax.axis_index("c")
    s = lax.axis_index("s")
    pltpu.sync_copy(src_hbm.at[s], src_v)
    pltpu.sync_copy(dst_hbm.at[s], dst_v)

    def zb(i, _):
        zero_v[i // 8, pl.ds((i % 8) * 16, 16)] = jnp.zeros((16,), jnp.float32)
        return 0

    lax.fori_loop(0, 128 * 8, zb, 0)
    row0 = pl.multiple_of(s * ROWS_PER_SUB, 8)

    for cc in range(2):
        chunk = c * 2 + cc
        for k in range(4):
            pltpu.sync_copy(zero_v, acc.at[pl.ds(row0 + k * 128, 128)])
        pltpu.sync_copy(zero_v.at[pl.ds(0, ROWS_PER_SUB - 512)],
                        acc.at[pl.ds(row0 + 512, ROWS_PER_SUB - 512)])

        def gx(i, _):
            r = i // 8
            k = i % 8
            gix_v[r, pl.ds(k * 16, 16)] = src_v[r, pl.ds(k * 16, 16)] * 4 + chunk
            return 0

        lax.fori_loop(0, NSTEP * 8, gx, 0)
        plsc.subcore_barrier()

        def step(j, _):
            pltpu.async_copy(h2d_hbm.at[gix_v.at[j]], rows_v, sem).wait()
            pltpu.sync_copy(rows_v, acc.at[dst_v.at[j]], add=True)
            return 0

        lax.fori_loop(0, NSTEP, step, 0)
        plsc.subcore_barrier()
        pltpu.sync_copy(acc.at[pl.ds(row0, ROWS_PER_SUB)],
                        out_hbm.at[chunk, pl.ds(row0, ROWS_PER_SUB)])
        plsc.subcore_barrier()


def _sc_agg(h2d, src16, dst16):
    f = pl.kernel(
        _sc_agg_body,
        out_type=jax.ShapeDtypeStruct((4, N_PAD, 128), jnp.float32),
        mesh=_sc_mesh(),
        compiler_params=pltpu.CompilerParams(needs_layout_passes=False),
        scratch_types=[
            pltpu.VMEM((NSTEP, 128), jnp.int32),
            pltpu.VMEM((NSTEP, 128), jnp.int32),
            pltpu.VMEM((NSTEP, 128), jnp.int32),
            pltpu.VMEM((128, 128), jnp.float32),
            pltpu.VMEM((128, 128), jnp.float32),
            pltpu.VMEM_SHARED((N_PAD, 128), jnp.float32),
            pltpu.SemaphoreType.DMA,
        ],
    )
    return f(h2d, src16, dst16)


def _dinv_body(hist_ref, out_ref):
    deg = jnp.sum(hist_ref[...], axis=0, keepdims=True) + 1.0
    out_ref[...] = lax.rsqrt(deg)


def _dinv(hist):
    return pl.pallas_call(
        _dinv_body,
        out_shape=jax.ShapeDtypeStruct((1, N_PAD), jnp.float32),
    )(hist)


def _mm1_body(x_ref, w_ref, dinv_ref, h1p_ref, pre2_ref):
    acc = jnp.dot(x_ref[...], w_ref[...], preferred_element_type=jnp.float32)
    h1p_ref[...] = acc[:, :H] * dinv_ref[...]
    pre2_ref[...] = acc[:, H:]


def _mm1(x_res, w12, dinv_col):
    return pl.pallas_call(
        _mm1_body,
        grid=(NRB,),
        in_specs=[
            pl.BlockSpec((RB, D_IN), lambda i: (i, 0)),
            pl.BlockSpec((D_IN, 2 * H), lambda i: (0, 0)),
            pl.BlockSpec((RB, 1), lambda i: (i, 0)),
        ],
        out_specs=[
            pl.BlockSpec((RB, H), lambda i: (i, 0)),
            pl.BlockSpec((RB, H), lambda i: (i, 0)),
        ],
        out_shape=[
            jax.ShapeDtypeStruct((N, H), jnp.float32),
            jax.ShapeDtypeStruct((N, H), jnp.float32),
        ],
    )(x_res, w12, dinv_col)


def _mm2_body(agg_ref, h1p_ref, dinv_ref, w2a_ref, pre2_ref,
              bc1_ref, g1_ref, be1_ref, h2p_ref, xx_ref):
    agg = jnp.concatenate([agg_ref[c] for c in range(4)], axis=1)
    out1 = (agg + h1p_ref[...]) * dinv_ref[...] + bc1_ref[...]
    s1 = g1_ref[...] * lax.rsqrt(jnp.float32(1.0 + BN_EPS))
    xx = jnp.maximum(out1, 0.0) * s1 + be1_ref[...]
    xx_ref[...] = xx
    h2 = jnp.dot(xx, w2a_ref[...], preferred_element_type=jnp.float32)
    h2p_ref[...] = (h2 + pre2_ref[...]) * dinv_ref[...]


def _mm2(agg1, h1p, dinv_col, w2a, pre2, bc1, g1, be1):
    return pl.pallas_call(
        _mm2_body,
        grid=(NRB,),
        in_specs=[
            pl.BlockSpec((4, RB, 128), lambda i: (0, i, 0)),
            pl.BlockSpec((RB, H), lambda i: (i, 0)),
            pl.BlockSpec((RB, 1), lambda i: (i, 0)),
            pl.BlockSpec((H, H), lambda i: (0, 0)),
            pl.BlockSpec((RB, H), lambda i: (i, 0)),
            pl.BlockSpec((1, H), lambda i: (0, 0)),
            pl.BlockSpec((1, H), lambda i: (0, 0)),
            pl.BlockSpec((1, H), lambda i: (0, 0)),
        ],
        out_specs=[
            pl.BlockSpec((RB, H), lambda i: (i, 0)),
            pl.BlockSpec((RB, H), lambda i: (i, 0)),
        ],
        out_shape=[
            jax.ShapeDtypeStruct((N, H), jnp.float32),
            jax.ShapeDtypeStruct((N, H), jnp.float32),
        ],
    )(agg1, h1p, dinv_col, w2a, pre2, bc1, g1, be1)


def _seg_body(agg_ref, h2p_ref, xx_ref, xres_ref, dinv_ref, batch_ref,
              bc2_ref, g2_ref, be2_ref, sxx_ref, sres_ref, sxxx_ref):
    i = pl.program_id(0)
    agg = jnp.concatenate([agg_ref[c] for c in range(4)], axis=1)
    out2 = (agg + h2p_ref[...]) * dinv_ref[...] + bc2_ref[...]
    s2 = g2_ref[...] * lax.rsqrt(jnp.float32(1.0 + BN_EPS))
    xxx = jnp.maximum(out2, 0.0) * s2 + be2_ref[...]

    rvalid = (lax.broadcasted_iota(jnp.int32, (RB, 1), 0) + i * RB) < N
    xxx = jnp.where(rvalid, xxx, 0.0)
    xx = jnp.where(rvalid, xx_ref[...], 0.0)
    xres = jnp.where(rvalid, xres_ref[...], 0.0)

    cvalid = (lax.broadcasted_iota(jnp.int32, (1, RB), 1) + i * RB) < N
    m = (batch_ref[...] == lax.broadcasted_iota(jnp.int32, (G, RB), 0)) & cvalid
    m = m.astype(jnp.float32)

    @pl.when(i == 0)
    def _():
        sxx_ref[...] = jnp.zeros_like(sxx_ref)
        sres_ref[...] = jnp.zeros_like(sres_ref)
        sxxx_ref[...] = jnp.zeros_like(sxxx_ref)

    sxx_ref[...] += jnp.dot(m, xx, preferred_element_type=jnp.float32)
    sres_ref[...] += jnp.dot(m, xres, preferred_element_type=jnp.float32)
    sxxx_ref[...] += jnp.dot(m, xxx, preferred_element_type=jnp.float32)


def _seg(agg2, h2p, xx, x_res, dinv_col, batch_row, bc2, g2, be2):
    return pl.pallas_call(
        _seg_body,
        grid=(NRB,),
        in_specs=[
            pl.BlockSpec((4, RB, 128), lambda i: (0, i, 0)),
            pl.BlockSpec((RB, H), lambda i: (i, 0)),
            pl.BlockSpec((RB, H), lambda i: (i, 0)),
            pl.BlockSpec((RB, D_IN), lambda i: (i, 0)),
            pl.BlockSpec((RB, 1), lambda i: (i, 0)),
            pl.BlockSpec((1, RB), lambda i: (0, i)),
            pl.BlockSpec((1, H), lambda i: (0, 0)),
            pl.BlockSpec((1, H), lambda i: (0, 0)),
            pl.BlockSpec((1, H), lambda i: (0, 0)),
        ],
        out_specs=[
            pl.BlockSpec((G, H), lambda i: (0, 0)),
            pl.BlockSpec((G, D_IN), lambda i: (0, 0)),
            pl.BlockSpec((G, H), lambda i: (0, 0)),
        ],
        out_shape=[
            jax.ShapeDtypeStruct((G, H), jnp.float32),
            jax.ShapeDtypeStruct((G, D_IN), jnp.float32),
            jax.ShapeDtypeStruct((G, H), jnp.float32),
        ],
    )(agg2, h2p, xx, x_res, dinv_col, batch_row, bc2, g2, be2)


def _fc1_body(xc_ref, xseq_ref, w_ref, b_ref, g4_ref, be4_ref, t_ref):
    y = jnp.dot(xseq_ref[...], w_ref[...], preferred_element_type=jnp.float32)
    x = xc_ref[...] + y + b_ref[...]
    s4 = g4_ref[...] * lax.rsqrt(jnp.float32(1.0 + BN_EPS))
    t_ref[...] = x * s4 + be4_ref[...]


def _fc1(xc_sum, x_seq, fc1w, fc1b, g4, be4):
    d_cat = 3584
    return pl.pallas_call(
        _fc1_body,
        grid=(7,),
        in_specs=[
            pl.BlockSpec((G, 512), lambda j: (0, j)),
            pl.BlockSpec((G, D_IN), lambda j: (0, 0)),
            pl.BlockSpec((D_IN, 512), lambda j: (0, j)),
            pl.BlockSpec((1, 512), lambda j: (0, j)),
            pl.BlockSpec((1, 512), lambda j: (0, j)),
            pl.BlockSpec((1, 512), lambda j: (0, j)),
        ],
        out_specs=pl.BlockSpec((G, 512), lambda j: (0, j)),
        out_shape=jax.ShapeDtypeStruct((G, d_cat), jnp.float32),
    )(xc_sum, x_seq, fc1w, fc1b, g4, be4)


def _fc2_body(t_ref, w_ref, b_ref, g5_ref, be5_ref, x1_ref):
    y = jnp.dot(t_ref[...], w_ref[...], preferred_element_type=jnp.float32)
    s5 = g5_ref[...] * lax.rsqrt(jnp.float32(1.0 + BN_EPS))
    x1_ref[...] = (y + b_ref[...]) * s5 + be5_ref[...]


def _fc2(t, fc2w, fc2b, g5, be5):
    d_cat = 3584
    return pl.pallas_call(
        _fc2_body,
        grid=(6,),
        in_specs=[
            pl.BlockSpec((G, d_cat), lambda j: (0, 0)),
            pl.BlockSpec((d_cat, 512), lambda j: (0, j)),
            pl.BlockSpec((1, 512), lambda j: (0, j)),
            pl.BlockSpec((1, 512), lambda j: (0, j)),
            pl.BlockSpec((1, 512), lambda j: (0, j)),
        ],
        out_specs=pl.BlockSpec((G, 512), lambda j: (0, j)),
        out_shape=jax.ShapeDtypeStruct((G, 3000), jnp.float32),
    )(t, fc2w, fc2b, g5, be5)


def _fc3_body(x1_ref, w_ref, b_ref, g6_ref, be6_ref, out_ref):
    wf = w_ref[0] + w_ref[1]
    y = jnp.dot(x1_ref[...], wf, preferred_element_type=jnp.float32)
    s6 = g6_ref[...] * lax.rsqrt(jnp.float32(1.0 + BN_EPS))
    out_ref[...] = jax.nn.sigmoid((y + b_ref[...]) * s6 + be6_ref[...])


def _fc3(x1, fc3w2, fc3b, g6, be6):
    return pl.pallas_call(
        _fc3_body,
        out_shape=jax.ShapeDtypeStruct((G, 500), jnp.float32),
    )(x1, fc3w2, fc3b, g6, be6)


def kernel(embedding_features_per_residue, embedding_features_per_sequence,
           sequence_features, edge_index, edge_attr, batch,
           W1, bc1, g1, be1, W2, bc2, g2, be2,
           fc1W, fc1b, g4, be4, fc2W, fc2b, g5, be5,
           fc3W, fc3b, g6, be6):
    x_res = embedding_features_per_residue
    x_seq = embedding_features_per_sequence

    src = edge_index[0]
    dst = edge_index[1]
    srcp = jnp.concatenate([src, jnp.zeros((E_PAD - E,), src.dtype)])
    dstp = jnp.concatenate([dst, jnp.full((E_PAD - E,), N, dst.dtype)])
    src16 = srcp.reshape(16, NSTEP, 128)
    dst16 = dstp.reshape(16, NSTEP, 128)
    dst32 = dstp.reshape(32, EPW_D // 128, 128)

    hist = _sc_deg(dst32).reshape(32, N_PAD)
    dinv_col = _dinv(hist).reshape(N_PAD, 1)

    w12 = jnp.concatenate([W1, W2[H:, :]], axis=1)
    h1p, pre2 = _mm1(x_res, w12, dinv_col)
    agg1 = _sc_agg(h1p.reshape(N * 4, 128), src16, dst16)

    r1 = lambda v: v.reshape(1, -1)
    h2p, xx = _mm2(agg1, h1p, dinv_col, W2[:H, :], pre2,
                   r1(bc1), r1(g1), r1(be1))
    agg2 = _sc_agg(h2p.reshape(N * 4, 128), src16, dst16)

    s_xx, s_res, s_xxx = _seg(agg2, h2p, xx, x_res, dinv_col,
                              batch.reshape(1, N), r1(bc2), r1(g2), r1(be2))
    xc_sum = jnp.concatenate([s_xx, s_res, s_xxx, s_res], axis=1)

    t = _fc1(xc_sum, x_seq, fc1W, r1(fc1b), r1(g4), r1(be4))
    x1 = _fc2(t, fc2W, r1(fc2b), r1(g5), r1(be5))
    out = _fc3(x1, fc3W.reshape(2, 3000, 500), r1(fc3b), r1(g6), r1(be6))
    return out
